# baseline (speedup 1.0000x reference)
import functools

import jax
import jax.numpy as jnp
from jax import lax
from jax.experimental import pallas as pl
from jax.experimental.pallas import tpu as pltpu
from jax.experimental.pallas import tpu_sc as plsc

N_FIELDS = 26
VOCAB = 100000
EMB = 32
B = 4096
D0 = N_FIELDS * EMB
H1 = 256
H2 = 128
NCLS = 10
EPS = 1e-5

NC = 2
NS = 16
NW = NC * NS
ROWS = B * N_FIELDS
CHUNK = 128
CHUNKS_PER_W = ROWS // (NW * CHUNK)


def _sc_gather(table_flat, idx_flat):
    mesh = plsc.VectorSubcoreMesh(core_axis_name="c", subcore_axis_name="s")

    @functools.partial(
        pl.kernel,
        mesh=mesh,
        compiler_params=pltpu.CompilerParams(use_tc_tiling_on_sc=False),
        out_type=jax.ShapeDtypeStruct((ROWS // CHUNK, CHUNK, EMB), jnp.float32),
        scratch_types=[
            pltpu.VMEM((CHUNKS_PER_W * CHUNK,), jnp.int32),
            pltpu.VMEM((CHUNKS_PER_W, CHUNK, EMB), jnp.float32),
            pltpu.SemaphoreType.DMA,
        ],
    )
    def k(table_hbm, idx_hbm, out_hbm, idx_v, rows_v, sem):
        wid = lax.axis_index("s") * NC + lax.axis_index("c")
        base = wid * CHUNKS_PER_W
        pltpu.sync_copy(idx_hbm.at[pl.ds(base * CHUNK, CHUNKS_PER_W * CHUNK)], idx_v)
        copies = [
            pltpu.async_copy(
                table_hbm.at[idx_v.at[pl.ds(j * CHUNK, CHUNK)]], rows_v.at[j], sem)
            for j in range(CHUNKS_PER_W)
        ]
        for c in copies:
            c.wait()
        pltpu.sync_copy(rows_v, out_hbm.at[pl.ds(base, CHUNKS_PER_W)])

    return k(table_flat, idx_flat)


def _mlp_body(z_ref, g0, b0, w1, b1, g1, bb1, w2, b2, g2, bb2, w3, b3, out_ref):
    def bn(x, g, b):
        mu = jnp.mean(x, axis=0, keepdims=True)
        var = jnp.mean((x - mu) * (x - mu), axis=0, keepdims=True)
        return (x - mu) * lax.rsqrt(var + EPS) * g[...] + b[...]

    z = bn(z_ref[...], g0, b0)
    h = jnp.dot(z, w1[...], preferred_element_type=jnp.float32,
                precision=lax.Precision.HIGHEST)
    h = jnp.maximum(h + b1[...], 0.0)
    h = bn(h, g1, bb1)
    h = jnp.dot(h, w2[...], preferred_element_type=jnp.float32,
                precision=lax.Precision.HIGHEST)
    h = jnp.maximum(h + b2[...], 0.0)
    h = bn(h, g2, bb2)
    out = jnp.dot(h, w3[...], preferred_element_type=jnp.float32,
                  precision=lax.Precision.HIGHEST)
    out_ref[...] = out + b3[...]


def kernel(x_cat, tables, bn0_g, bn0_b, W1, b1, bn1_g, bn1_b, W2, b2, bn2_g, bn2_b, W3, b3):
    offs = (jnp.arange(N_FIELDS, dtype=jnp.int32) * VOCAB)[None, :]
    idx = (x_cat.astype(jnp.int32) + offs).reshape(ROWS)
    table_flat = tables.reshape(N_FIELDS * VOCAB, EMB)

    rows = _sc_gather(table_flat, idx)
    z = rows.reshape(B, D0)

    out = pl.pallas_call(
        _mlp_body,
        out_shape=jax.ShapeDtypeStruct((B, NCLS), jnp.float32),
    )(
        z,
        bn0_g.reshape(1, D0), bn0_b.reshape(1, D0),
        W1, b1.reshape(1, H1), bn1_g.reshape(1, H1), bn1_b.reshape(1, H1),
        W2, b2.reshape(1, H2), bn2_g.reshape(1, H2), bn2_b.reshape(1, H2),
        W3, b3.reshape(1, NCLS),
    )
    return out
